# baseline (device time: 15438 ns/iter reference)
import jax
import jax.numpy as jnp
from jax import lax
from jax.experimental import pallas as pl
from jax.experimental.pallas import tpu as pltpu

N_DEV = 16
N_TOK = 512
D_IN = 256
D_OUT = 512
N_EXP = 64
E_LOCAL = 4
CAP = 6
C_ROWS = E_LOCAL * CAP
G_ROWS = N_DEV * C_ROWS
ROW_CHUNK = 128

HIGH = lax.Precision.HIGHEST


def _body(x_ref, ridx_ref, w_ref, out_ref, g_ref, send_sems, recv_sems,
          ready_sems):
    my = lax.axis_index("i")

    barrier = pltpu.get_barrier_semaphore()
    pl.semaphore_signal(barrier, 1)
    pl.semaphore_wait(barrier, 1)

    for off in range(1, N_DEV):
        dst = lax.rem(my + N_DEV - off, N_DEV)
        pl.semaphore_signal(ready_sems.at[off - 1], 1, device_id=(dst,),
                            device_id_type=pl.DeviceIdType.MESH)

    ridx = ridx_ref[:, :]
    eid = lax.broadcasted_iota(jnp.int32, (N_TOK, N_EXP), 1)
    onehot = (ridx == eid).astype(jnp.float32)
    row = lax.broadcasted_iota(jnp.int32, (N_TOK, N_TOK), 0)
    col = lax.broadcasted_iota(jnp.int32, (N_TOK, N_TOK), 1)
    lower = (col <= row).astype(jnp.float32)
    ranks = jnp.dot(lower, onehot, preferred_element_type=jnp.float32)
    rank = jnp.sum(ranks * onehot, axis=1, keepdims=True)

    cslot = lax.broadcasted_iota(jnp.int32, (N_TOK, C_ROWS), 1)
    slot_exp = my * E_LOCAL + cslot // CAP
    slot_rank = (cslot % CAP + 1).astype(jnp.float32)
    T = ((ridx == slot_exp) & (rank == slot_rank)).astype(jnp.float32)
    Xc = lax.dot_general(T, x_ref[:, :], (((0,), (0,)), ((), ())),
                         precision=HIGH,
                         preferred_element_type=jnp.float32)

    crow = lax.broadcasted_iota(jnp.int32, (C_ROWS, 1), 0)
    C = jnp.zeros((C_ROWS, D_OUT), jnp.float32)
    for k in range(E_LOCAL):
        m = (crow // CAP == k).astype(jnp.float32)
        C = C + m * jnp.dot(Xc, w_ref[k], precision=HIGH,
                            preferred_element_type=jnp.float32)
    g_ref[pl.ds(my * C_ROWS, C_ROWS), :] = C.astype(jnp.bfloat16)

    sends = []
    for off in range(1, N_DEV):
        dst = lax.rem(my + off, N_DEV)
        pl.semaphore_wait(ready_sems.at[off - 1], 1)
        rd = pltpu.make_async_remote_copy(
            src_ref=g_ref.at[pl.ds(my * C_ROWS, C_ROWS), :],
            dst_ref=g_ref.at[pl.ds(my * C_ROWS, C_ROWS), :],
            send_sem=send_sems.at[off - 1],
            recv_sem=recv_sems.at[off - 1],
            device_id=(dst,),
            device_id_type=pl.DeviceIdType.MESH,
        )
        rd.start()
        sends.append(rd)

    gcol = lax.broadcasted_iota(jnp.int32, (ROW_CHUNK, G_ROWS), 1)
    g_exp = gcol // CAP
    g_rank = (gcol % CAP + 1).astype(jnp.float32)
    S_chunks = []
    for rc in range(N_TOK // ROW_CHUNK):
        sl = slice(rc * ROW_CHUNK, (rc + 1) * ROW_CHUNK)
        S_chunks.append(
            ((ridx[sl] == g_exp) & (rank[sl] == g_rank)).astype(jnp.bfloat16)
        )

    for off in range(1, N_DEV):
        src = lax.rem(my + N_DEV - off, N_DEV)
        rd = pltpu.make_async_remote_copy(
            src_ref=g_ref.at[pl.ds(src * C_ROWS, C_ROWS), :],
            dst_ref=g_ref.at[pl.ds(src * C_ROWS, C_ROWS), :],
            send_sem=send_sems.at[off - 1],
            recv_sem=recv_sems.at[off - 1],
            device_id=(src,),
            device_id_type=pl.DeviceIdType.MESH,
        )
        rd.wait_recv()

    G = g_ref[:, :]
    for rc in range(N_TOK // ROW_CHUNK):
        sl = slice(rc * ROW_CHUNK, (rc + 1) * ROW_CHUNK)
        out_ref[sl, :] = jnp.dot(S_chunks[rc], G,
                                 preferred_element_type=jnp.float32)

    for rd in sends:
        rd.wait_send()


def kernel(x, router_W, route_idx, expert_W):
    del router_W
    return pl.pallas_call(
        _body,
        out_shape=jax.ShapeDtypeStruct((N_TOK, D_OUT), jnp.float32),
        in_specs=[
            pl.BlockSpec(memory_space=pltpu.VMEM),
            pl.BlockSpec(memory_space=pltpu.VMEM),
            pl.BlockSpec(memory_space=pltpu.VMEM),
        ],
        out_specs=pl.BlockSpec(memory_space=pltpu.VMEM),
        scratch_shapes=[
            pltpu.VMEM((G_ROWS, D_OUT), jnp.bfloat16),
            pltpu.SemaphoreType.DMA((N_DEV - 1,)),
            pltpu.SemaphoreType.DMA((N_DEV - 1,)),
            pltpu.SemaphoreType.REGULAR((N_DEV - 1,)),
        ],
        compiler_params=pltpu.CompilerParams(collective_id=0),
    )(x, route_idx, expert_W)


# device time: 15281 ns/iter; 1.0103x vs baseline; 1.0103x over previous
import jax
import jax.numpy as jnp
from jax import lax
from jax.experimental import pallas as pl
from jax.experimental.pallas import tpu as pltpu

N_DEV = 16
N_TOK = 512
D_IN = 256
D_OUT = 512
N_EXP = 64
E_LOCAL = 4
CAP = 6
C_ROWS = E_LOCAL * CAP
G_ROWS = N_DEV * C_ROWS
ROW_CHUNK = 128

HIGH = lax.Precision.HIGHEST


def _body(x_ref, ridx_ref, w_ref, out_ref, g_ref, send_sems, recv_sems):
    my = lax.axis_index("i")

    barrier = pltpu.get_barrier_semaphore()
    for off in range(1, N_DEV):
        dst = lax.rem(my + off, N_DEV)
        pl.semaphore_signal(barrier, 1, device_id=(dst,),
                            device_id_type=pl.DeviceIdType.MESH)

    ridx = ridx_ref[:, :]
    eid = lax.broadcasted_iota(jnp.int32, (N_TOK, N_EXP), 1)
    onehot = (ridx == eid).astype(jnp.float32)
    row = lax.broadcasted_iota(jnp.int32, (N_TOK, N_TOK), 0)
    col = lax.broadcasted_iota(jnp.int32, (N_TOK, N_TOK), 1)
    lower = (col <= row).astype(jnp.float32)
    ranks = jnp.dot(lower, onehot, preferred_element_type=jnp.float32)
    rank = jnp.sum(ranks * onehot, axis=1, keepdims=True)

    cslot = lax.broadcasted_iota(jnp.int32, (N_TOK, C_ROWS), 1)
    slot_exp = my * E_LOCAL + cslot // CAP
    slot_rank = (cslot % CAP + 1).astype(jnp.float32)
    T = ((ridx == slot_exp) & (rank == slot_rank)).astype(jnp.float32)
    Xc = lax.dot_general(T, x_ref[:, :], (((0,), (0,)), ((), ())),
                         precision=HIGH,
                         preferred_element_type=jnp.float32)

    crow = lax.broadcasted_iota(jnp.int32, (C_ROWS, 1), 0)
    C = jnp.zeros((C_ROWS, D_OUT), jnp.float32)
    for k in range(E_LOCAL):
        m = (crow // CAP == k).astype(jnp.float32)
        C = C + m * jnp.dot(Xc, w_ref[k], precision=HIGH,
                            preferred_element_type=jnp.float32)
    g_ref[pl.ds(my * C_ROWS, C_ROWS), :] = C.astype(jnp.bfloat16)

    pl.semaphore_wait(barrier, N_DEV - 1)

    sends = []
    for off in range(1, N_DEV):
        dst = lax.rem(my + off, N_DEV)
        rd = pltpu.make_async_remote_copy(
            src_ref=g_ref.at[pl.ds(my * C_ROWS, C_ROWS), :],
            dst_ref=g_ref.at[pl.ds(my * C_ROWS, C_ROWS), :],
            send_sem=send_sems.at[off - 1],
            recv_sem=recv_sems.at[off - 1],
            device_id=(dst,),
            device_id_type=pl.DeviceIdType.MESH,
        )
        rd.start()
        sends.append(rd)

    gcol = lax.broadcasted_iota(jnp.int32, (ROW_CHUNK, G_ROWS), 1)
    g_exp = gcol // CAP
    g_rank = (gcol % CAP + 1).astype(jnp.float32)
    S_chunks = []
    for rc in range(N_TOK // ROW_CHUNK):
        sl = slice(rc * ROW_CHUNK, (rc + 1) * ROW_CHUNK)
        S_chunks.append(
            ((ridx[sl] == g_exp) & (rank[sl] == g_rank)).astype(jnp.bfloat16)
        )

    for off in range(1, N_DEV):
        src = lax.rem(my + N_DEV - off, N_DEV)
        rd = pltpu.make_async_remote_copy(
            src_ref=g_ref.at[pl.ds(src * C_ROWS, C_ROWS), :],
            dst_ref=g_ref.at[pl.ds(src * C_ROWS, C_ROWS), :],
            send_sem=send_sems.at[off - 1],
            recv_sem=recv_sems.at[off - 1],
            device_id=(src,),
            device_id_type=pl.DeviceIdType.MESH,
        )
        rd.wait_recv()

    G = g_ref[:, :]
    for rc in range(N_TOK // ROW_CHUNK):
        sl = slice(rc * ROW_CHUNK, (rc + 1) * ROW_CHUNK)
        out_ref[sl, :] = jnp.dot(S_chunks[rc], G,
                                 preferred_element_type=jnp.float32)

    for rd in sends:
        rd.wait_send()


def kernel(x, router_W, route_idx, expert_W):
    del router_W
    return pl.pallas_call(
        _body,
        out_shape=jax.ShapeDtypeStruct((N_TOK, D_OUT), jnp.float32),
        in_specs=[
            pl.BlockSpec(memory_space=pltpu.VMEM),
            pl.BlockSpec(memory_space=pltpu.VMEM),
            pl.BlockSpec(memory_space=pltpu.VMEM),
        ],
        out_specs=pl.BlockSpec(memory_space=pltpu.VMEM),
        scratch_shapes=[
            pltpu.VMEM((G_ROWS, D_OUT), jnp.bfloat16),
            pltpu.SemaphoreType.DMA((N_DEV - 1,)),
            pltpu.SemaphoreType.DMA((N_DEV - 1,)),
        ],
        compiler_params=pltpu.CompilerParams(collective_id=0),
    )(x, route_idx, expert_W)
